# baseline (speedup 1.0000x reference)
import functools

import jax
import jax.numpy as jnp
from jax import lax
from jax.experimental import pallas as pl
from jax.experimental.pallas import tpu as pltpu
from jax.experimental.pallas import tpu_sc as plsc

_IN_DIM = 101
_D = 64
_BATCH = 16384
_HIST = 200
_MAX_NORM = 1.0

_B = _BATCH * _HIST
_IDXW = 128
_NROWS = _B // _IDXW

_NC = 2
_NS = 16
_NW = _NC * _NS
_ROWS_PW = _NROWS // _NW

_NSUB = 5
_C = _NSUB * _IDXW
_G = _ROWS_PW // _NSUB

_IDX_BYTES = _NSUB * _IDXW * 4
_ROWS_BYTES = _C * _D * 4


def _norm_body(tab_ref, out_ref):
    t = tab_ref[...]
    norms = jnp.sqrt(jnp.sum(t * t, axis=1, keepdims=True))
    scale = jnp.where(norms > _MAX_NORM, _MAX_NORM / (norms + 1e-7), 1.0)
    out_ref[...] = t * scale


def _normalize_table(table):
    return pl.pallas_call(
        _norm_body,
        out_shape=jax.ShapeDtypeStruct((_IN_DIM, _D), jnp.float32),
    )(table)


def _gather_body(tab_hbm, idx_hbm, out_hbm, idx_v, rows_v, sems):
    isem0, isem1, gsem0, gsem1, ssem0, ssem1 = sems
    wid = lax.axis_index("s") * _NC + lax.axis_index("c")
    row_base = wid * _ROWS_PW
    out_base = wid * _ROWS_PW * _IDXW
    last_row = _NROWS - _NSUB

    def idx_start(g, slot_v, isem):
        off = lax.min(row_base + g * _NSUB, last_row)
        pltpu.async_copy(idx_hbm.at[pl.ds(off, _NSUB)], slot_v, isem)

    def store_wait(rows, ssem):
        pltpu.make_async_copy(rows, out_hbm.at[pl.ds(out_base, _C)], ssem).wait()

    def phase(g, slot_v, rows, isem, gsem, ssem, first=False):
        if not first:
            store_wait(rows, ssem)
        pltpu.make_async_copy(
            idx_hbm.at[pl.ds(row_base, _NSUB)], slot_v, isem
        ).wait()
        cps = [
            pltpu.async_copy(
                tab_hbm.at[slot_v.at[j]],
                rows.at[pl.ds(j * _IDXW, _IDXW)],
                gsem,
            )
            for j in range(_NSUB)
        ]
        for cp in cps:
            cp.wait()
        idx_start(g + 2, slot_v, isem)
        pltpu.async_copy(rows, out_hbm.at[pl.ds(out_base + g * _C, _C)], ssem)

    idx_start(0, idx_v.at[0], isem0)
    idx_start(1, idx_v.at[1], isem1)
    phase(0, idx_v.at[0], rows_v.at[0], isem0, gsem0, ssem0, first=True)
    phase(1, idx_v.at[1], rows_v.at[1], isem1, gsem1, ssem1, first=True)

    def pair(gp, carry):
        g0 = gp * 2
        phase(g0, idx_v.at[0], rows_v.at[0], isem0, gsem0, ssem0)
        phase(g0 + 1, idx_v.at[1], rows_v.at[1], isem1, gsem1, ssem1)
        return carry

    lax.fori_loop(1, _G // 2, pair, 0)

    store_wait(rows_v.at[0], ssem0)
    store_wait(rows_v.at[1], ssem1)
    pltpu.make_async_copy(
        idx_hbm.at[pl.ds(row_base, _NSUB)], idx_v.at[0], isem0
    ).wait()
    pltpu.make_async_copy(
        idx_hbm.at[pl.ds(row_base, _NSUB)], idx_v.at[1], isem1
    ).wait()


@functools.partial(
    pl.kernel,
    out_type=jax.ShapeDtypeStruct((_B, _D), jnp.float32),
    mesh=plsc.VectorSubcoreMesh(core_axis_name="c", subcore_axis_name="s"),
    scratch_types=[
        pltpu.VMEM((2, _NSUB, _IDXW), jnp.int32),
        pltpu.VMEM((2, _C, _D), jnp.float32),
        pltpu.SemaphoreType.DMA,
        pltpu.SemaphoreType.DMA,
        pltpu.SemaphoreType.DMA,
        pltpu.SemaphoreType.DMA,
        pltpu.SemaphoreType.DMA,
        pltpu.SemaphoreType.DMA,
    ],
    compiler_params=pltpu.CompilerParams(use_tc_tiling_on_sc=False),
)
def _sc_gather(tab_hbm, idx_hbm, out_hbm, idx_v, rows_v, *sems):
    _gather_body(tab_hbm, idx_hbm, out_hbm, idx_v, rows_v, sems)


def kernel(x, table):
    norm_tab = _normalize_table(table)
    idx = x.reshape(_NROWS, _IDXW)
    flat = _sc_gather(norm_tab, idx)
    return flat.reshape(_BATCH, _HIST, _D)


# speedup vs baseline: 1.0035x; 1.0035x over previous
import functools

import jax
import jax.numpy as jnp
from jax import lax
from jax.experimental import pallas as pl
from jax.experimental.pallas import tpu as pltpu
from jax.experimental.pallas import tpu_sc as plsc

_IN_DIM = 101
_D = 64
_BATCH = 16384
_HIST = 200
_MAX_NORM = 1.0

_B = _BATCH * _HIST
_IDXW = 800
_NROWS = _B // _IDXW

_NC = 2
_NS = 16
_NW = _NC * _NS
_ROWS_PW = _NROWS // _NW

_NSUB = 1
_C = _NSUB * _IDXW
_G = _ROWS_PW // _NSUB

_IDX_BYTES = _NSUB * _IDXW * 4
_ROWS_BYTES = _C * _D * 4


def _norm_body(tab_ref, out_ref):
    t = tab_ref[...]
    norms = jnp.sqrt(jnp.sum(t * t, axis=1, keepdims=True))
    scale = jnp.where(norms > _MAX_NORM, _MAX_NORM / (norms + 1e-7), 1.0)
    out_ref[...] = t * scale


def _normalize_table(table):
    return pl.pallas_call(
        _norm_body,
        out_shape=jax.ShapeDtypeStruct((_IN_DIM, _D), jnp.float32),
    )(table)


def _gather_body(tab_hbm, idx_hbm, out_hbm, idx_v, rows_v, sems):
    isem0, isem1, gsem0, gsem1, ssem0, ssem1 = sems
    wid = lax.axis_index("s") * _NC + lax.axis_index("c")
    row_base = wid * _ROWS_PW
    out_base = wid * _ROWS_PW * _IDXW
    last_row = _NROWS - _NSUB

    def idx_start(g, slot_v, isem):
        off = lax.min(row_base + g * _NSUB, last_row)
        pltpu.async_copy(idx_hbm.at[pl.ds(off, _NSUB)], slot_v, isem)

    def store_wait(rows, ssem):
        pltpu.make_async_copy(rows, out_hbm.at[pl.ds(out_base, _C)], ssem).wait()

    def phase(g, slot_v, rows, isem, gsem, ssem, first=False):
        if not first:
            store_wait(rows, ssem)
        pltpu.make_async_copy(
            idx_hbm.at[pl.ds(row_base, _NSUB)], slot_v, isem
        ).wait()
        cps = [
            pltpu.async_copy(
                tab_hbm.at[slot_v.at[j]],
                rows.at[pl.ds(j * _IDXW, _IDXW)],
                gsem,
            )
            for j in range(_NSUB)
        ]
        for cp in cps:
            cp.wait()
        idx_start(g + 2, slot_v, isem)
        pltpu.async_copy(rows, out_hbm.at[pl.ds(out_base + g * _C, _C)], ssem)

    idx_start(0, idx_v.at[0], isem0)
    idx_start(1, idx_v.at[1], isem1)
    phase(0, idx_v.at[0], rows_v.at[0], isem0, gsem0, ssem0, first=True)
    phase(1, idx_v.at[1], rows_v.at[1], isem1, gsem1, ssem1, first=True)

    def pair(gp, carry):
        g0 = gp * 2
        phase(g0, idx_v.at[0], rows_v.at[0], isem0, gsem0, ssem0)
        phase(g0 + 1, idx_v.at[1], rows_v.at[1], isem1, gsem1, ssem1)
        return carry

    lax.fori_loop(1, _G // 2, pair, 0)

    store_wait(rows_v.at[0], ssem0)
    store_wait(rows_v.at[1], ssem1)
    pltpu.make_async_copy(
        idx_hbm.at[pl.ds(row_base, _NSUB)], idx_v.at[0], isem0
    ).wait()
    pltpu.make_async_copy(
        idx_hbm.at[pl.ds(row_base, _NSUB)], idx_v.at[1], isem1
    ).wait()


@functools.partial(
    pl.kernel,
    out_type=jax.ShapeDtypeStruct((_B, _D), jnp.float32),
    mesh=plsc.VectorSubcoreMesh(core_axis_name="c", subcore_axis_name="s"),
    scratch_types=[
        pltpu.VMEM((2, _NSUB, _IDXW), jnp.int32),
        pltpu.VMEM((2, _C, _D), jnp.float32),
        pltpu.SemaphoreType.DMA,
        pltpu.SemaphoreType.DMA,
        pltpu.SemaphoreType.DMA,
        pltpu.SemaphoreType.DMA,
        pltpu.SemaphoreType.DMA,
        pltpu.SemaphoreType.DMA,
    ],
    compiler_params=pltpu.CompilerParams(use_tc_tiling_on_sc=False),
)
def _sc_gather(tab_hbm, idx_hbm, out_hbm, idx_v, rows_v, *sems):
    _gather_body(tab_hbm, idx_hbm, out_hbm, idx_v, rows_v, sems)


def kernel(x, table):
    norm_tab = _normalize_table(table)
    idx = x.reshape(_NROWS, _IDXW)
    flat = _sc_gather(norm_tab, idx)
    return flat.reshape(_BATCH, _HIST, _D)


# speedup vs baseline: 1.8952x; 1.8886x over previous
import functools

import jax
import jax.numpy as jnp
from jax import lax
from jax.experimental import pallas as pl
from jax.experimental.pallas import tpu as pltpu
from jax.experimental.pallas import tpu_sc as plsc

_IN_DIM = 101
_D = 64
_BATCH = 16384
_HIST = 200
_MAX_NORM = 1.0

_B = _BATCH * _HIST
_IDXW = 800
_NROWS = _B // _IDXW

_NC = 2
_NS = 16
_NW = _NC * _NS
_ROWS_PW = _NROWS // _NW

_NSUB = 1
_C = _NSUB * _IDXW
_G = _ROWS_PW // _NSUB

_IDX_BYTES = _NSUB * _IDXW * 4
_ROWS_BYTES = _C * _D * 4


def _norm_body(tab_ref, out_ref):
    t = tab_ref[...]
    norms = jnp.sqrt(jnp.sum(t * t, axis=1, keepdims=True))
    scale = jnp.where(norms > _MAX_NORM, _MAX_NORM / (norms + 1e-7), 1.0)
    out_ref[...] = t * scale


def _normalize_table(table):
    return pl.pallas_call(
        _norm_body,
        out_shape=jax.ShapeDtypeStruct((_IN_DIM, _D), jnp.float32),
    )(table)


def _gather_body(tab_hbm, idx_hbm, out_hbm, tab_v, idx_v, rows_v, sems):
    isem0, isem1, gsem0, gsem1, ssem0, ssem1 = sems
    sid = lax.axis_index("s")
    @pl.when(sid == 0)
    def _():
        pltpu.sync_copy(tab_hbm, tab_v)
    plsc.subcore_barrier()
    wid = lax.axis_index("s") * _NC + lax.axis_index("c")
    row_base = wid * _ROWS_PW
    out_base = wid * _ROWS_PW * _IDXW
    last_row = _NROWS - _NSUB

    def idx_start(g, slot_v, isem):
        off = lax.min(row_base + g * _NSUB, last_row)
        pltpu.async_copy(idx_hbm.at[pl.ds(off, _NSUB)], slot_v, isem)

    def store_wait(rows, ssem):
        pltpu.make_async_copy(rows, out_hbm.at[pl.ds(out_base, _C)], ssem).wait()

    def phase(g, slot_v, rows, isem, gsem, ssem, first=False):
        if not first:
            store_wait(rows, ssem)
        pltpu.make_async_copy(
            idx_hbm.at[pl.ds(row_base, _NSUB)], slot_v, isem
        ).wait()
        cps = [
            pltpu.async_copy(
                tab_v.at[slot_v.at[j]],
                rows.at[pl.ds(j * _IDXW, _IDXW)],
                gsem,
            )
            for j in range(_NSUB)
        ]
        for cp in cps:
            cp.wait()
        idx_start(g + 2, slot_v, isem)
        pltpu.async_copy(rows, out_hbm.at[pl.ds(out_base + g * _C, _C)], ssem)

    idx_start(0, idx_v.at[0], isem0)
    idx_start(1, idx_v.at[1], isem1)
    phase(0, idx_v.at[0], rows_v.at[0], isem0, gsem0, ssem0, first=True)
    phase(1, idx_v.at[1], rows_v.at[1], isem1, gsem1, ssem1, first=True)

    def pair(gp, carry):
        g0 = gp * 2
        phase(g0, idx_v.at[0], rows_v.at[0], isem0, gsem0, ssem0)
        phase(g0 + 1, idx_v.at[1], rows_v.at[1], isem1, gsem1, ssem1)
        return carry

    lax.fori_loop(1, _G // 2, pair, 0)

    store_wait(rows_v.at[0], ssem0)
    store_wait(rows_v.at[1], ssem1)
    pltpu.make_async_copy(
        idx_hbm.at[pl.ds(row_base, _NSUB)], idx_v.at[0], isem0
    ).wait()
    pltpu.make_async_copy(
        idx_hbm.at[pl.ds(row_base, _NSUB)], idx_v.at[1], isem1
    ).wait()


@functools.partial(
    pl.kernel,
    out_type=jax.ShapeDtypeStruct((_B, _D), jnp.float32),
    mesh=plsc.VectorSubcoreMesh(core_axis_name="c", subcore_axis_name="s"),
    scratch_types=[
        pltpu.VMEM_SHARED((_IN_DIM, _D), jnp.float32),
        pltpu.VMEM((2, _NSUB, _IDXW), jnp.int32),
        pltpu.VMEM((2, _C, _D), jnp.float32),
        pltpu.SemaphoreType.DMA,
        pltpu.SemaphoreType.DMA,
        pltpu.SemaphoreType.DMA,
        pltpu.SemaphoreType.DMA,
        pltpu.SemaphoreType.DMA,
        pltpu.SemaphoreType.DMA,
    ],
    compiler_params=pltpu.CompilerParams(use_tc_tiling_on_sc=False),
)
def _sc_gather(tab_hbm, idx_hbm, out_hbm, tab_v, idx_v, rows_v, *sems):
    _gather_body(tab_hbm, idx_hbm, out_hbm, tab_v, idx_v, rows_v, sems)


def kernel(x, table):
    norm_tab = _normalize_table(table)
    idx = x.reshape(_NROWS, _IDXW)
    flat = _sc_gather(norm_tab, idx)
    return flat.reshape(_BATCH, _HIST, _D)


# speedup vs baseline: 1.8961x; 1.0005x over previous
import functools

import jax
import jax.numpy as jnp
from jax import lax
from jax.experimental import pallas as pl
from jax.experimental.pallas import tpu as pltpu
from jax.experimental.pallas import tpu_sc as plsc

_IN_DIM = 101
_D = 64
_BATCH = 16384
_HIST = 200
_MAX_NORM = 1.0

_NC = 2
_NS = 16
_NW = _NC * _NS
_BAT_PW = _BATCH // _NW

_NSUB = 4
_C = _NSUB * _HIST
_G = _BAT_PW // _NSUB


def _norm_body(tab_ref, out_ref):
    t = tab_ref[...]
    norms = jnp.sqrt(jnp.sum(t * t, axis=1, keepdims=True))
    scale = jnp.where(norms > _MAX_NORM, _MAX_NORM / (norms + 1e-7), 1.0)
    out_ref[...] = t * scale


def _normalize_table(table):
    return pl.pallas_call(
        _norm_body,
        out_shape=jax.ShapeDtypeStruct((_IN_DIM, _D), jnp.float32),
    )(table)


def _gather_body(tab_hbm, idx_hbm, out_hbm, tab_s, idx_v, rows_v, sems):
    isem0, isem1, gsem0, gsem1, ssem0, ssem1 = sems
    sid = lax.axis_index("s")

    @pl.when(sid == 0)
    def _():
        pltpu.sync_copy(tab_hbm, tab_s)

    plsc.subcore_barrier()

    wid = sid * _NC + lax.axis_index("c")
    bat_base = wid * _BAT_PW
    last_bat = _BATCH - _NSUB

    def idx_start(g, slot_v, isem):
        off = lax.min(bat_base + g * _NSUB, last_bat)
        pltpu.async_copy(idx_hbm.at[pl.ds(off, _NSUB)], slot_v, isem)

    def store_wait(rows, ssem):
        pltpu.make_async_copy(
            rows, out_hbm.at[pl.ds(bat_base, _NSUB)], ssem
        ).wait()

    def phase(g, slot_v, rows, isem, gsem, ssem, first=False):
        if not first:
            store_wait(rows, ssem)
        pltpu.make_async_copy(
            idx_hbm.at[pl.ds(bat_base, _NSUB)], slot_v, isem
        ).wait()
        cps = [
            pltpu.async_copy(
                tab_s.at[slot_v.at[j]],
                rows.at[j],
                gsem,
            )
            for j in range(_NSUB)
        ]
        for cp in cps:
            cp.wait()
        idx_start(g + 2, slot_v, isem)
        pltpu.async_copy(
            rows, out_hbm.at[pl.ds(bat_base + g * _NSUB, _NSUB)], ssem
        )

    idx_start(0, idx_v.at[0], isem0)
    idx_start(1, idx_v.at[1], isem1)
    phase(0, idx_v.at[0], rows_v.at[0], isem0, gsem0, ssem0, first=True)
    phase(1, idx_v.at[1], rows_v.at[1], isem1, gsem1, ssem1, first=True)

    def pair(gp, carry):
        g0 = gp * 2
        phase(g0, idx_v.at[0], rows_v.at[0], isem0, gsem0, ssem0)
        phase(g0 + 1, idx_v.at[1], rows_v.at[1], isem1, gsem1, ssem1)
        return carry

    lax.fori_loop(1, _G // 2, pair, 0)

    store_wait(rows_v.at[0], ssem0)
    store_wait(rows_v.at[1], ssem1)
    pltpu.make_async_copy(
        idx_hbm.at[pl.ds(bat_base, _NSUB)], idx_v.at[0], isem0
    ).wait()
    pltpu.make_async_copy(
        idx_hbm.at[pl.ds(bat_base, _NSUB)], idx_v.at[1], isem1
    ).wait()


@functools.partial(
    pl.kernel,
    out_type=jax.ShapeDtypeStruct((_BATCH, _HIST, _D), jnp.float32),
    mesh=plsc.VectorSubcoreMesh(core_axis_name="c", subcore_axis_name="s"),
    scratch_types=[
        pltpu.VMEM_SHARED((_IN_DIM, _D), jnp.float32),
        pltpu.VMEM((2, _NSUB, _HIST), jnp.int32),
        pltpu.VMEM((2, _NSUB, _HIST, _D), jnp.float32),
        pltpu.SemaphoreType.DMA,
        pltpu.SemaphoreType.DMA,
        pltpu.SemaphoreType.DMA,
        pltpu.SemaphoreType.DMA,
        pltpu.SemaphoreType.DMA,
        pltpu.SemaphoreType.DMA,
    ],
    compiler_params=pltpu.CompilerParams(use_tc_tiling_on_sc=False),
)
def _sc_gather(tab_hbm, idx_hbm, out_hbm, tab_s, idx_v, rows_v, *sems):
    _gather_body(tab_hbm, idx_hbm, out_hbm, tab_s, idx_v, rows_v, sems)


def kernel(x, table):
    norm_tab = _normalize_table(table)
    return _sc_gather(norm_tab, x)
